# baseline (speedup 1.0000x reference)
import functools

import jax
import jax.numpy as jnp
from jax import lax
from jax.experimental import pallas as pl
from jax.experimental.pallas import tpu as pltpu
from jax.experimental.pallas import tpu_sc as plsc

T = 768
L = 96
B = 16
ENC = 256
POS_DIM = 64
REL = 256


def _sc_gathers(pos_lut_pad, src_flat, feats0, flat_idx):
    info = plsc.get_sparse_core_info()
    nw = info.num_cores * info.num_subcores
    rpw = T // nw
    mesh = plsc.VectorSubcoreMesh(core_axis_name="c", subcore_axis_name="s")

    @functools.partial(
        pl.kernel,
        out_type=(
            jax.ShapeDtypeStruct((T, 2 * POS_DIM), jnp.float32),
            jax.ShapeDtypeStruct((T, ENC), jnp.float32),
        ),
        mesh=mesh,
        scratch_types=[
            pltpu.VMEM((rpw,), jnp.int32),
            pltpu.VMEM((rpw,), jnp.int32),
            pltpu.VMEM((rpw, 2 * POS_DIM), jnp.float32),
            pltpu.VMEM((rpw, ENC), jnp.float32),
            pltpu.SemaphoreType.DMA,
            pltpu.SemaphoreType.DMA,
        ],
    )
    def gather_kernel(lut_hbm, src_hbm, f0_hbm, fi_hbm, psd_hbm, hs_hbm,
                      idx1, idx2, rows1, rows2, sem1, sem2):
        wid = lax.axis_index("s") * info.num_cores + lax.axis_index("c")
        base = wid * rpw
        pltpu.sync_copy(f0_hbm.at[pl.ds(base, rpw)], idx1)
        pltpu.sync_copy(fi_hbm.at[pl.ds(base, rpw)], idx2)
        c1 = pltpu.async_copy(lut_hbm.at[idx1], rows1, sem1)
        c2 = pltpu.async_copy(src_hbm.at[idx2], rows2, sem2)
        c1.wait()
        c2.wait()
        pltpu.sync_copy(rows1, psd_hbm.at[pl.ds(base, rpw)])
        pltpu.sync_copy(rows2, hs_hbm.at[pl.ds(base, rpw)])

    psd_pad, head_sel = gather_kernel(pos_lut_pad, src_flat, feats0, flat_idx)
    return psd_pad[:, :POS_DIM], head_sel


def _tc_dense_small(psd_emb, head_sel, pos_lut, W_head, b_head2, W_dep):
    def body(psd_ref, hs_ref, lut_ref, wh_ref, bh_ref, wd_ref, ho_ref, md_ref):
        ho_ref[...] = (
            jnp.dot(psd_ref[...], wh_ref[:POS_DIM, :],
                    preferred_element_type=jnp.float32)
            + jnp.dot(hs_ref[...], wh_ref[POS_DIM:, :],
                      preferred_element_type=jnp.float32)
            + bh_ref[...]
        )
        md_ref[...] = jnp.dot(lut_ref[...], wd_ref[:POS_DIM, :],
                              preferred_element_type=jnp.float32)

    return pl.pallas_call(
        body,
        out_shape=(
            jax.ShapeDtypeStruct((T, REL), jnp.float32),
            jax.ShapeDtypeStruct((POS_DIM, REL), jnp.float32),
        ),
    )(psd_emb, head_sel, pos_lut, W_head, b_head2, W_dep)


TB = 8


CH = 32
NCH = L // CH


def _tc_dep(seg, lens, maps, src_enc, idxTF, fsTF, M_dep, W_enc, b_dep2):
    def body(seg_ref, lens_ref, maps_ref, *refs):
        src_refs = refs[:NCH]
        idx_ref, fs_ref, md_ref, w_ref, b_ref, out_ref = refs[NCH:]
        tb = pl.program_id(0)
        cols_p = lax.broadcasted_iota(jnp.int32, (L, POS_DIM), 1)
        segs = lax.broadcasted_iota(jnp.int32, (B, 1), 0)
        rows = lax.broadcasted_iota(jnp.int32, (L, 1), 0)
        cols_ch = lax.broadcasted_iota(jnp.int32, (L, CH), 1)
        for i in range(TB):
            t = tb * TB + i
            oh = (segs == seg_ref[t]).astype(jnp.float32)
            idx_col = jnp.dot(idx_ref[...], oh,
                              preferred_element_type=jnp.float32)
            f_col = jnp.dot(fs_ref[...], oh,
                            preferred_element_type=jnp.float32)
            idx_i = idx_col.astype(jnp.int32)
            G = None
            for c in range(NCH):
                Pc = (idx_i == cols_ch + c * CH).astype(jnp.bfloat16)
                Gc = jnp.dot(Pc, src_refs[c][i].astype(jnp.bfloat16),
                             preferred_element_type=jnp.float32)
                G = Gc if G is None else G + Gc
            OF = (f_col.astype(jnp.int32) == cols_p).astype(jnp.bfloat16)
            acc = (
                jnp.dot(G.astype(jnp.bfloat16), w_ref[...],
                        preferred_element_type=jnp.float32)
                + jnp.dot(OF, md_ref[...].astype(jnp.bfloat16),
                          preferred_element_type=jnp.float32)
                + b_ref[...]
            )
            out_ref[i] = jnp.where(rows < lens_ref[t], acc, 0.0)

    def _chunk_spec(c):
        if c == 0:
            return pl.BlockSpec((TB, CH, ENC),
                                lambda t, seg_r, lens_r, maps_r: (t, 0, 0))
        return pl.BlockSpec(
            (TB, CH, ENC),
            lambda t, seg_r, lens_r, maps_r, c=c: (maps_r[c - 1, t], c, 0))

    grid_spec = pltpu.PrefetchScalarGridSpec(
        num_scalar_prefetch=3,
        grid=(T // TB,),
        in_specs=[_chunk_spec(c) for c in range(NCH)] + [
            pl.BlockSpec((L, B), lambda t, seg_r, lens_r, maps_r: (0, 0)),
            pl.BlockSpec((L, B), lambda t, seg_r, lens_r, maps_r: (0, 0)),
            pl.BlockSpec((POS_DIM, REL),
                         lambda t, seg_r, lens_r, maps_r: (0, 0)),
            pl.BlockSpec((ENC, REL), lambda t, seg_r, lens_r, maps_r: (0, 0)),
            pl.BlockSpec((1, REL), lambda t, seg_r, lens_r, maps_r: (0, 0)),
        ],
        out_specs=pl.BlockSpec((TB, L, ENC),
                               lambda t, seg_r, lens_r, maps_r: (t, 0, 0)),
    )
    return pl.pallas_call(
        body,
        grid_spec=grid_spec,
        out_shape=jax.ShapeDtypeStruct((T, L, REL), jnp.float32),
    )(seg, lens, maps, *([src_enc] * NCH), idxTF, fsTF, M_dep, W_enc, b_dep2)


def kernel(feats, lengths, index, src_enc, pos_lut, W_head, b_head, W_dep,
           b_dep):
    lengths = lengths.astype(jnp.int32)
    index = index.astype(jnp.int32)
    csum = jnp.cumsum(lengths)
    offsets = csum - lengths
    tpos = jnp.arange(T, dtype=jnp.int32)
    seg = jnp.searchsorted(csum, tpos, side="right").astype(jnp.int32)
    lens = lengths[seg]
    feats0 = feats[:, 0].astype(jnp.int32)
    flat_idx = tpos * L + index
    jj = jnp.arange(L, dtype=jnp.int32)
    fp = jnp.clip(offsets[:, None] + jj[None, :], 0, T - 1)
    idxTF = index[fp].T.astype(jnp.float32)
    fsTF = feats0[fp].T.astype(jnp.float32)

    pos_lut_pad = jnp.pad(pos_lut, ((0, 0), (0, 2 * POS_DIM - pos_lut.shape[1])))
    psd_emb, head_sel = _sc_gathers(pos_lut_pad, src_enc.reshape(T * L, ENC),
                                    feats0, flat_idx)
    head_out, M_dep = _tc_dense_small(psd_emb, head_sel, pos_lut, W_head,
                                      b_head.reshape(1, REL), W_dep)
    nt = T // TB
    maxlen_blk = lens.reshape(nt, TB).max(axis=1)
    blk_iota = jnp.arange(nt, dtype=jnp.int32)
    maps = jnp.stack([
        lax.cummax(jnp.where(maxlen_blk > c * CH, blk_iota, 0))
        for c in range(1, NCH)
    ])
    dep_out = _tc_dep(seg, lens, maps, src_enc, idxTF, fsTF, M_dep,
                      W_dep[POS_DIM:].astype(jnp.bfloat16),
                      b_dep.reshape(1, REL))
    return (psd_emb, head_out, dep_out)


# speedup vs baseline: 1.7587x; 1.7587x over previous
import functools

import jax
import jax.numpy as jnp
from jax import lax
from jax.experimental import pallas as pl
from jax.experimental.pallas import tpu as pltpu
from jax.experimental.pallas import tpu_sc as plsc

T = 768
L = 96
B = 16
ENC = 256
POS_DIM = 64
REL = 256


def _sc_gathers(pos_lut_pad, src_flat, feats0, flat_idx):
    info = plsc.get_sparse_core_info()
    nw = info.num_cores * info.num_subcores
    rpw = T // nw
    mesh = plsc.VectorSubcoreMesh(core_axis_name="c", subcore_axis_name="s")

    @functools.partial(
        pl.kernel,
        out_type=(
            jax.ShapeDtypeStruct((T, 2 * POS_DIM), jnp.float32),
            jax.ShapeDtypeStruct((T, ENC), jnp.float32),
        ),
        mesh=mesh,
        scratch_types=[
            pltpu.VMEM((rpw,), jnp.int32),
            pltpu.VMEM((rpw,), jnp.int32),
            pltpu.VMEM((rpw, 2 * POS_DIM), jnp.float32),
            pltpu.VMEM((rpw, ENC), jnp.float32),
            pltpu.SemaphoreType.DMA,
            pltpu.SemaphoreType.DMA,
        ],
    )
    def gather_kernel(lut_hbm, src_hbm, f0_hbm, fi_hbm, psd_hbm, hs_hbm,
                      idx1, idx2, rows1, rows2, sem1, sem2):
        wid = lax.axis_index("s") * info.num_cores + lax.axis_index("c")
        base = wid * rpw
        pltpu.sync_copy(f0_hbm.at[pl.ds(base, rpw)], idx1)
        pltpu.sync_copy(fi_hbm.at[pl.ds(base, rpw)], idx2)
        c1 = pltpu.async_copy(lut_hbm.at[idx1], rows1, sem1)
        c2 = pltpu.async_copy(src_hbm.at[idx2], rows2, sem2)
        c1.wait()
        c2.wait()
        pltpu.sync_copy(rows1, psd_hbm.at[pl.ds(base, rpw)])
        pltpu.sync_copy(rows2, hs_hbm.at[pl.ds(base, rpw)])

    psd_pad, head_sel = gather_kernel(pos_lut_pad, src_flat, feats0, flat_idx)
    return psd_pad[:, :POS_DIM], head_sel


def _tc_dense_small(psd_emb, head_sel, pos_lut, W_head, b_head2, W_dep):
    def body(psd_ref, hs_ref, lut_ref, wh_ref, bh_ref, wd_ref, ho_ref, md_ref):
        ho_ref[...] = (
            jnp.dot(psd_ref[...], wh_ref[:POS_DIM, :],
                    preferred_element_type=jnp.float32)
            + jnp.dot(hs_ref[...], wh_ref[POS_DIM:, :],
                      preferred_element_type=jnp.float32)
            + bh_ref[...]
        )
        md_ref[...] = jnp.dot(lut_ref[...], wd_ref[:POS_DIM, :],
                              preferred_element_type=jnp.float32)

    return pl.pallas_call(
        body,
        out_shape=(
            jax.ShapeDtypeStruct((T, REL), jnp.float32),
            jax.ShapeDtypeStruct((POS_DIM, REL), jnp.float32),
        ),
    )(psd_emb, head_sel, pos_lut, W_head, b_head2, W_dep)


TB = 16


def _tc_dep(seg, lens, src_enc, idxTF, fsTF, M_dep, W_enc, b_dep2):
    def body(seg_ref, lens_ref, src_ref, idx_ref, fs_ref, md_ref, w_ref,
             b_ref, out_ref):
        tb = pl.program_id(0)
        cols_l = lax.broadcasted_iota(jnp.int32, (L, L), 1)
        cols_p = lax.broadcasted_iota(jnp.int32, (L, POS_DIM), 1)
        segs = lax.broadcasted_iota(jnp.int32, (B, 1), 0)
        rows = lax.broadcasted_iota(jnp.int32, (L, 1), 0)
        for i in range(TB):
            t = tb * TB + i
            oh = (segs == seg_ref[t]).astype(jnp.float32)
            idx_col = jnp.dot(idx_ref[...], oh,
                              preferred_element_type=jnp.float32)
            f_col = jnp.dot(fs_ref[...], oh,
                            preferred_element_type=jnp.float32)
            P = (idx_col.astype(jnp.int32) == cols_l).astype(jnp.float32)
            OF = (f_col.astype(jnp.int32) == cols_p).astype(jnp.float32)
            G = jnp.dot(P, src_ref[i], preferred_element_type=jnp.float32)
            acc = (
                jnp.dot(G, w_ref[...], preferred_element_type=jnp.float32)
                + jnp.dot(OF, md_ref[...], preferred_element_type=jnp.float32)
                + b_ref[...]
            )
            out_ref[i] = jnp.where(rows < lens_ref[t], acc, 0.0)

    grid_spec = pltpu.PrefetchScalarGridSpec(
        num_scalar_prefetch=2,
        grid=(T // TB,),
        in_specs=[
            pl.BlockSpec((TB, L, ENC), lambda t, seg_r, lens_r: (t, 0, 0)),
            pl.BlockSpec((L, B), lambda t, seg_r, lens_r: (0, 0)),
            pl.BlockSpec((L, B), lambda t, seg_r, lens_r: (0, 0)),
            pl.BlockSpec((POS_DIM, REL), lambda t, seg_r, lens_r: (0, 0)),
            pl.BlockSpec((ENC, REL), lambda t, seg_r, lens_r: (0, 0)),
            pl.BlockSpec((1, REL), lambda t, seg_r, lens_r: (0, 0)),
        ],
        out_specs=pl.BlockSpec((TB, L, ENC),
                               lambda t, seg_r, lens_r: (t, 0, 0)),
    )
    return pl.pallas_call(
        body,
        grid_spec=grid_spec,
        out_shape=jax.ShapeDtypeStruct((T, L, REL), jnp.float32),
    )(seg, lens, src_enc, idxTF, fsTF, M_dep, W_enc, b_dep2)


def kernel(feats, lengths, index, src_enc, pos_lut, W_head, b_head, W_dep,
           b_dep):
    lengths = lengths.astype(jnp.int32)
    index = index.astype(jnp.int32)
    csum = jnp.cumsum(lengths)
    offsets = csum - lengths
    tpos = jnp.arange(T, dtype=jnp.int32)
    seg = jnp.searchsorted(csum, tpos, side="right").astype(jnp.int32)
    lens = lengths[seg]
    feats0 = feats[:, 0].astype(jnp.int32)
    flat_idx = tpos * L + index
    jj = jnp.arange(L, dtype=jnp.int32)
    fp = jnp.clip(offsets[:, None] + jj[None, :], 0, T - 1)
    idxTF = index[fp].T.astype(jnp.float32)
    fsTF = feats0[fp].T.astype(jnp.float32)

    pos_lut_pad = jnp.pad(pos_lut, ((0, 0), (0, 2 * POS_DIM - pos_lut.shape[1])))
    psd_emb, head_sel = _sc_gathers(pos_lut_pad, src_enc.reshape(T * L, ENC),
                                    feats0, flat_idx)
    head_out, M_dep = _tc_dense_small(psd_emb, head_sel, pos_lut, W_head,
                                      b_head.reshape(1, REL), W_dep)
    dep_out = _tc_dep(seg, lens, src_enc, idxTF, fsTF, M_dep,
                      W_dep[POS_DIM:], b_dep.reshape(1, REL))
    return (psd_emb, head_out, dep_out)


# speedup vs baseline: 2.7215x; 1.5475x over previous
import functools

import jax
import jax.numpy as jnp
from jax import lax
from jax.experimental import pallas as pl
from jax.experimental.pallas import tpu as pltpu
from jax.experimental.pallas import tpu_sc as plsc

T = 768
L = 96
B = 16
ENC = 256
POS_DIM = 64
REL = 256


def _sc_gathers(pos_lut_pad, src_flat, feats0, flat_idx):
    info = plsc.get_sparse_core_info()
    nw = info.num_cores * info.num_subcores
    rpw = T // nw
    mesh = plsc.VectorSubcoreMesh(core_axis_name="c", subcore_axis_name="s")

    @functools.partial(
        pl.kernel,
        out_type=(
            jax.ShapeDtypeStruct((T, 2 * POS_DIM), jnp.float32),
            jax.ShapeDtypeStruct((T, ENC), jnp.float32),
        ),
        mesh=mesh,
        scratch_types=[
            pltpu.VMEM((rpw,), jnp.int32),
            pltpu.VMEM((rpw,), jnp.int32),
            pltpu.VMEM((rpw, 2 * POS_DIM), jnp.float32),
            pltpu.VMEM((rpw, ENC), jnp.float32),
            pltpu.SemaphoreType.DMA,
            pltpu.SemaphoreType.DMA,
        ],
    )
    def gather_kernel(lut_hbm, src_hbm, f0_hbm, fi_hbm, psd_hbm, hs_hbm,
                      idx1, idx2, rows1, rows2, sem1, sem2):
        wid = lax.axis_index("s") * info.num_cores + lax.axis_index("c")
        base = wid * rpw
        pltpu.sync_copy(f0_hbm.at[pl.ds(base, rpw)], idx1)
        pltpu.sync_copy(fi_hbm.at[pl.ds(base, rpw)], idx2)
        c1 = pltpu.async_copy(lut_hbm.at[idx1], rows1, sem1)
        c2 = pltpu.async_copy(src_hbm.at[idx2], rows2, sem2)
        c1.wait()
        c2.wait()
        pltpu.sync_copy(rows1, psd_hbm.at[pl.ds(base, rpw)])
        pltpu.sync_copy(rows2, hs_hbm.at[pl.ds(base, rpw)])

    psd_pad, head_sel = gather_kernel(pos_lut_pad, src_flat, feats0, flat_idx)
    return psd_pad[:, :POS_DIM], head_sel


def _tc_dense_small(psd_emb, head_sel, pos_lut, W_head, b_head2, W_dep):
    def body(psd_ref, hs_ref, lut_ref, wh_ref, bh_ref, wd_ref, ho_ref, md_ref):
        ho_ref[...] = (
            jnp.dot(psd_ref[...], wh_ref[:POS_DIM, :],
                    preferred_element_type=jnp.float32)
            + jnp.dot(hs_ref[...], wh_ref[POS_DIM:, :],
                      preferred_element_type=jnp.float32)
            + bh_ref[...]
        )
        md_ref[...] = jnp.dot(lut_ref[...], wd_ref[:POS_DIM, :],
                              preferred_element_type=jnp.float32)

    return pl.pallas_call(
        body,
        out_shape=(
            jax.ShapeDtypeStruct((T, REL), jnp.float32),
            jax.ShapeDtypeStruct((POS_DIM, REL), jnp.float32),
        ),
    )(psd_emb, head_sel, pos_lut, W_head, b_head2, W_dep)


TB = 48


def _tc_dep(seg, lens, segb, src_enc, idxTF, fsTF, M_dep, W_enc, b_dep2):
    def body(seg_ref, lens_ref, segb_ref, src_ref, idx_ref, fs_ref, md_ref,
             w_ref, b_ref, out_ref, g_scr, of_scr):
        tb = pl.program_id(0)
        cols_l = lax.broadcasted_iota(jnp.int32, (L, L), 1)
        cols_p = lax.broadcasted_iota(jnp.int32, (L, POS_DIM), 1)
        rows = lax.broadcasted_iota(jnp.int32, (L, 1), 0)
        iota_b = lax.broadcasted_iota(jnp.int32, (B, TB), 0)
        seg_row = segb_ref[0][0:1, :]
        OH = (jnp.broadcast_to(seg_row, (B, TB)) == iota_b).astype(jnp.float32)
        IDX = jnp.dot(idx_ref[...], OH,
                      preferred_element_type=jnp.float32).astype(jnp.int32)
        F = jnp.dot(fs_ref[...], OH,
                    preferred_element_type=jnp.float32).astype(jnp.int32)
        for i in range(TB):
            P = (IDX[:, i:i + 1] == cols_l).astype(jnp.float32)
            g_scr[pl.ds(i * L, L), :] = jnp.dot(
                P, src_ref[i], preferred_element_type=jnp.float32)
            of_scr[pl.ds(i * L, L), :] = (
                F[:, i:i + 1] == cols_p).astype(jnp.float32)
        acc = (
            jnp.dot(g_scr[...], w_ref[...], preferred_element_type=jnp.float32)
            + jnp.dot(of_scr[...], md_ref[...],
                      preferred_element_type=jnp.float32)
            + b_ref[...]
        )
        for i in range(TB):
            out_ref[i] = jnp.where(rows < lens_ref[tb * TB + i],
                                   acc[i * L:(i + 1) * L], 0.0)

    grid_spec = pltpu.PrefetchScalarGridSpec(
        num_scalar_prefetch=2,
        grid=(T // TB,),
        in_specs=[
            pl.BlockSpec((1, 8, TB), lambda t, seg_r, lens_r: (t, 0, 0)),
            pl.BlockSpec((TB, L, ENC), lambda t, seg_r, lens_r: (t, 0, 0)),
            pl.BlockSpec((L, B), lambda t, seg_r, lens_r: (0, 0)),
            pl.BlockSpec((L, B), lambda t, seg_r, lens_r: (0, 0)),
            pl.BlockSpec((POS_DIM, REL), lambda t, seg_r, lens_r: (0, 0)),
            pl.BlockSpec((ENC, REL), lambda t, seg_r, lens_r: (0, 0)),
            pl.BlockSpec((1, REL), lambda t, seg_r, lens_r: (0, 0)),
        ],
        out_specs=pl.BlockSpec((TB, L, ENC),
                               lambda t, seg_r, lens_r: (t, 0, 0)),
        scratch_shapes=[
            pltpu.VMEM((TB * L, ENC), jnp.float32),
            pltpu.VMEM((TB * L, POS_DIM), jnp.float32),
        ],
    )
    return pl.pallas_call(
        body,
        grid_spec=grid_spec,
        out_shape=jax.ShapeDtypeStruct((T, L, REL), jnp.float32),
    )(seg, lens, segb, src_enc, idxTF, fsTF, M_dep, W_enc, b_dep2)


def kernel(feats, lengths, index, src_enc, pos_lut, W_head, b_head, W_dep,
           b_dep):
    lengths = lengths.astype(jnp.int32)
    index = index.astype(jnp.int32)
    csum = jnp.cumsum(lengths)
    offsets = csum - lengths
    tpos = jnp.arange(T, dtype=jnp.int32)
    seg = jnp.searchsorted(csum, tpos, side="right").astype(jnp.int32)
    lens = lengths[seg]
    feats0 = feats[:, 0].astype(jnp.int32)
    flat_idx = tpos * L + index
    jj = jnp.arange(L, dtype=jnp.int32)
    fp = jnp.clip(offsets[:, None] + jj[None, :], 0, T - 1)
    idxTF = index[fp].T.astype(jnp.float32)
    fsTF = feats0[fp].T.astype(jnp.float32)

    pos_lut_pad = jnp.pad(pos_lut, ((0, 0), (0, 2 * POS_DIM - pos_lut.shape[1])))
    psd_emb, head_sel = _sc_gathers(pos_lut_pad, src_enc.reshape(T * L, ENC),
                                    feats0, flat_idx)
    head_out, M_dep = _tc_dense_small(psd_emb, head_sel, pos_lut, W_head,
                                      b_head.reshape(1, REL), W_dep)
    segb = jnp.broadcast_to(seg.reshape(T // TB, 1, TB),
                            (T // TB, 8, TB))
    dep_out = _tc_dep(seg, lens, segb, src_enc, idxTF, fsTF, M_dep,
                      W_dep[POS_DIM:], b_dep.reshape(1, REL))
    return (psd_emb, head_out, dep_out)


# speedup vs baseline: 2.7783x; 1.0209x over previous
import functools

import jax
import jax.numpy as jnp
from jax import lax
from jax.experimental import pallas as pl
from jax.experimental.pallas import tpu as pltpu
from jax.experimental.pallas import tpu_sc as plsc

T = 768
L = 96
B = 16
ENC = 256
POS_DIM = 64
REL = 256


def _sc_gathers(pos_lut_pad, src_flat, feats0, flat_idx):
    info = plsc.get_sparse_core_info()
    nw = info.num_cores * info.num_subcores
    rpw = T // nw
    mesh = plsc.VectorSubcoreMesh(core_axis_name="c", subcore_axis_name="s")

    @functools.partial(
        pl.kernel,
        out_type=(
            jax.ShapeDtypeStruct((T, 2 * POS_DIM), jnp.float32),
            jax.ShapeDtypeStruct((T, ENC), jnp.float32),
        ),
        mesh=mesh,
        scratch_types=[
            pltpu.VMEM((rpw,), jnp.int32),
            pltpu.VMEM((rpw,), jnp.int32),
            pltpu.VMEM((rpw, 2 * POS_DIM), jnp.float32),
            pltpu.VMEM((rpw, ENC), jnp.float32),
            pltpu.SemaphoreType.DMA,
            pltpu.SemaphoreType.DMA,
        ],
    )
    def gather_kernel(lut_hbm, src_hbm, f0_hbm, fi_hbm, psd_hbm, hs_hbm,
                      idx1, idx2, rows1, rows2, sem1, sem2):
        wid = lax.axis_index("s") * info.num_cores + lax.axis_index("c")
        base = wid * rpw
        pltpu.sync_copy(f0_hbm.at[pl.ds(base, rpw)], idx1)
        pltpu.sync_copy(fi_hbm.at[pl.ds(base, rpw)], idx2)
        c1 = pltpu.async_copy(lut_hbm.at[idx1], rows1, sem1)
        c2 = pltpu.async_copy(src_hbm.at[idx2], rows2, sem2)
        c1.wait()
        c2.wait()
        pltpu.sync_copy(rows1, psd_hbm.at[pl.ds(base, rpw)])
        pltpu.sync_copy(rows2, hs_hbm.at[pl.ds(base, rpw)])

    psd_pad, head_sel = gather_kernel(pos_lut_pad, src_flat, feats0, flat_idx)
    return psd_pad[:, :POS_DIM], head_sel


def _tc_dense_small(psd_emb, head_sel, W_head, b_head2):
    def body(psd_ref, hs_ref, wh_ref, bh_ref, ho_ref):
        ho_ref[...] = (
            jnp.dot(psd_ref[...], wh_ref[:POS_DIM, :],
                    preferred_element_type=jnp.float32)
            + jnp.dot(hs_ref[...], wh_ref[POS_DIM:, :],
                      preferred_element_type=jnp.float32)
            + bh_ref[...]
        )

    return pl.pallas_call(
        body,
        out_shape=jax.ShapeDtypeStruct((T, REL), jnp.float32),
    )(psd_emb, head_sel, W_head, b_head2)


TB = 96


def _tc_dep(seg, lens, segb, src_enc, idxTF, fsTF, pos_lut, W_dep_pos, W_enc,
            b_dep2):
    def body(seg_ref, lens_ref, segb_ref, src_ref, idx_ref, fs_ref, lut_ref,
             wdp_ref, w_ref, b_ref, out_ref, g_scr, of_scr):
        tb = pl.program_id(0)
        md = jnp.dot(lut_ref[...], wdp_ref[...],
                     preferred_element_type=jnp.float32)
        cols_l = lax.broadcasted_iota(jnp.int32, (L, L), 1)
        cols_p = lax.broadcasted_iota(jnp.int32, (L, POS_DIM), 1)
        rows = lax.broadcasted_iota(jnp.int32, (L, 1), 0)
        iota_b = lax.broadcasted_iota(jnp.int32, (B, TB), 0)
        seg_row = segb_ref[0][0:1, :]
        OH = (jnp.broadcast_to(seg_row, (B, TB)) == iota_b).astype(jnp.float32)
        IDX = jnp.dot(idx_ref[...], OH,
                      preferred_element_type=jnp.float32).astype(jnp.int32)
        F = jnp.dot(fs_ref[...], OH,
                    preferred_element_type=jnp.float32).astype(jnp.int32)
        for i in range(TB):
            P = (IDX[:, i:i + 1] == cols_l).astype(jnp.float32)
            g_scr[pl.ds(i * L, L), :] = jnp.dot(
                P, src_ref[i], preferred_element_type=jnp.float32)
            of_scr[pl.ds(i * L, L), :] = (
                F[:, i:i + 1] == cols_p).astype(jnp.float32)
        acc = (
            jnp.dot(g_scr[...], w_ref[...], preferred_element_type=jnp.float32)
            + jnp.dot(of_scr[...], md,
                      preferred_element_type=jnp.float32)
            + b_ref[...]
        )
        for i in range(TB):
            out_ref[i] = jnp.where(rows < lens_ref[tb * TB + i],
                                   acc[i * L:(i + 1) * L], 0.0)

    grid_spec = pltpu.PrefetchScalarGridSpec(
        num_scalar_prefetch=2,
        grid=(T // TB,),
        in_specs=[
            pl.BlockSpec((1, 8, TB), lambda t, seg_r, lens_r: (t, 0, 0)),
            pl.BlockSpec((TB, L, ENC), lambda t, seg_r, lens_r: (t, 0, 0)),
            pl.BlockSpec((L, B), lambda t, seg_r, lens_r: (0, 0)),
            pl.BlockSpec((L, B), lambda t, seg_r, lens_r: (0, 0)),
            pl.BlockSpec((POS_DIM, POS_DIM), lambda t, seg_r, lens_r: (0, 0)),
            pl.BlockSpec((POS_DIM, REL), lambda t, seg_r, lens_r: (0, 0)),
            pl.BlockSpec((ENC, REL), lambda t, seg_r, lens_r: (0, 0)),
            pl.BlockSpec((1, REL), lambda t, seg_r, lens_r: (0, 0)),
        ],
        out_specs=pl.BlockSpec((TB, L, ENC),
                               lambda t, seg_r, lens_r: (t, 0, 0)),
        scratch_shapes=[
            pltpu.VMEM((TB * L, ENC), jnp.float32),
            pltpu.VMEM((TB * L, POS_DIM), jnp.float32),
        ],
    )
    return pl.pallas_call(
        body,
        grid_spec=grid_spec,
        out_shape=jax.ShapeDtypeStruct((T, L, REL), jnp.float32),
    )(seg, lens, segb, src_enc, idxTF, fsTF, pos_lut, W_dep_pos, W_enc,
      b_dep2)


def kernel(feats, lengths, index, src_enc, pos_lut, W_head, b_head, W_dep,
           b_dep):
    lengths = lengths.astype(jnp.int32)
    index = index.astype(jnp.int32)
    csum = jnp.cumsum(lengths)
    offsets = csum - lengths
    tpos = jnp.arange(T, dtype=jnp.int32)
    seg = jnp.searchsorted(csum, tpos, side="right").astype(jnp.int32)
    lens = lengths[seg]
    feats0 = feats[:, 0].astype(jnp.int32)
    flat_idx = tpos * L + index
    jj = jnp.arange(L, dtype=jnp.int32)
    fp = jnp.clip(offsets[:, None] + jj[None, :], 0, T - 1)
    idxTF = index[fp].T.astype(jnp.float32)
    fsTF = feats0[fp].T.astype(jnp.float32)

    pos_lut_pad = jnp.pad(pos_lut, ((0, 0), (0, 2 * POS_DIM - pos_lut.shape[1])))
    psd_emb, head_sel = _sc_gathers(pos_lut_pad, src_enc.reshape(T * L, ENC),
                                    feats0, flat_idx)
    segb = jnp.broadcast_to(seg.reshape(T // TB, 1, TB),
                            (T // TB, 8, TB))
    dep_out = _tc_dep(seg, lens, segb, src_enc, idxTF, fsTF,
                      pos_lut, W_dep[:POS_DIM], W_dep[POS_DIM:],
                      b_dep.reshape(1, REL))
    head_out = _tc_dense_small(psd_emb, head_sel, W_head,
                               b_head.reshape(1, REL))
    return (psd_emb, head_out, dep_out)
